# Initial kernel scaffold; baseline (speedup 1.0000x reference)
import jax
import jax.numpy as jnp
from jax.experimental import pallas as pl


def kernel(x, edge_index, W_l_0, W_r_0, b_0, W_l_1, W_r_1, b_1, W_l_2, W_r_2, b_2, W_l_3, W_r_3, b_3, W_l_4, W_r_4, b_4, W_l_5, W_r_5, b_5):
    raise NotImplementedError("write your pallas kernel here")


# speedup vs baseline: 14.3441x; 14.3441x over previous
import functools

import jax
import jax.numpy as jnp
from jax import lax
from jax.experimental import pallas as pl
from jax.experimental.pallas import tpu as pltpu
from jax.experimental.pallas import tpu_sc as plsc

N = 10000
D = 128
NC, NS = 2, 16
NW = NC * NS
N_PAD = 10240
RPT = N_PAD // NS
NG = 12288
GPW = NG // NW

_MESH = dict(core_axis_name="c", subcore_axis_name="s", num_cores=NC,
             num_subcores=NS)


def _agg_body(zs, srcs, dsts, zeros2, out, acc, srcb, dstb, rows4, gsem):
    c = lax.axis_index("c")
    s = lax.axis_index("s")
    w = c * NS + s
    r0 = s * RPT
    pltpu.sync_copy(zeros2.at[pl.ds(r0, RPT)], acc.at[pl.ds(r0, RPT)])
    plsc.subcore_barrier()
    nblk = srcs.shape[0] // NW // 256

    def blk(i, carry):
        e0 = (w * nblk + i) * 256
        for k in range(2):
            pltpu.sync_copy(srcs.at[pl.ds(e0 + k * 128, 128)], srcb.at[k])
            pltpu.sync_copy(dsts.at[pl.ds(e0 + k * 128, 128)], dstb.at[k])
        descs = [pltpu.async_copy(zs.at[srcb.at[k]], rows4.at[k], gsem)
                 for k in range(2)]
        for d in descs:
            d.wait()
        for k in range(2):
            pltpu.sync_copy(rows4.at[k], acc.at[dstb.at[k]], add=True)
        return carry

    lax.fori_loop(0, nblk, blk, 0)
    plsc.subcore_barrier()
    pltpu.sync_copy(acc.at[pl.ds(r0, RPT)],
                    out.at[pl.ds(c * N_PAD + r0, RPT)])


def _make_agg(ep):
    return functools.partial(
        pl.kernel,
        out_type=jax.ShapeDtypeStruct((NC * N_PAD, D), jnp.float32),
        mesh=plsc.VectorSubcoreMesh(**_MESH),
        scratch_types=[
            pltpu.VMEM_SHARED((N_PAD, D), jnp.float32),
            pltpu.VMEM((2, 128), jnp.int32),
            pltpu.VMEM((2, 128), jnp.int32),
            pltpu.VMEM((2, 128, D), jnp.float32),
            pltpu.SemaphoreType.DMA,
        ],
    )(_agg_body)


def _counts_body(dsts, zeros1, ones1, out, cacc, dstb, onesb):
    c = lax.axis_index("c")
    s = lax.axis_index("s")
    w = c * NS + s
    r0 = s * RPT
    pltpu.sync_copy(zeros1.at[pl.ds(r0, RPT)], cacc.at[pl.ds(r0, RPT)])
    pltpu.sync_copy(ones1, onesb)
    plsc.subcore_barrier()
    nblk = dsts.shape[0] // NW // 512

    def blk(i, carry):
        e0 = (w * nblk + i) * 512
        for k in range(4):
            pltpu.sync_copy(dsts.at[pl.ds(e0 + k * 128, 128)], dstb.at[k])
        for k in range(4):
            pltpu.sync_copy(onesb, cacc.at[dstb.at[k]], add=True)
        return carry

    lax.fori_loop(0, nblk, blk, 0)
    plsc.subcore_barrier()
    pltpu.sync_copy(cacc.at[pl.ds(r0, RPT)],
                    out.at[pl.ds(c * N_PAD + r0, RPT)])


def _make_counts():
    return functools.partial(
        pl.kernel,
        out_type=jax.ShapeDtypeStruct((NC * N_PAD,), jnp.float32),
        mesh=plsc.VectorSubcoreMesh(**_MESH),
        scratch_types=[
            pltpu.VMEM_SHARED((N_PAD,), jnp.float32),
            pltpu.VMEM((4, 128), jnp.int32),
            pltpu.VMEM((128,), jnp.float32),
        ],
    )(_counts_body)


def _gather8_body(xs, y1s, z1s, y2s, z2s, y3s, c1, c2, c3, p1, p2, uu,
                  oA, oB, oC, oD, oF, oG, oH, oU, idxb, rows, gsem):
    c = lax.axis_index("c")
    s = lax.axis_index("s")
    w = c * NS + s
    specs = [(y1s, c1, oA), (z1s, c2, oB), (y2s, c2, oC), (z2s, c3, oD),
             (y3s, c3, oF), (xs, p1, oG), (xs, p2, oH), (y3s, uu, oU)]
    for src_ref, idx_ref, out_ref in specs:
        for ch in range(GPW // 128):
            pltpu.sync_copy(
                idx_ref.at[pl.ds(w * GPW + ch * 128, 128)], idxb.at[ch])
        for ch in range(GPW // 128):
            pltpu.async_copy(src_ref.at[idxb.at[ch]], rows, gsem).wait()
            pltpu.sync_copy(
                rows, out_ref.at[pl.ds(w * GPW + ch * 128, 128)])


def _make_gather8():
    ot = [jax.ShapeDtypeStruct((NG, D), jnp.float32) for _ in range(8)]
    return functools.partial(
        pl.kernel,
        out_type=ot,
        mesh=plsc.VectorSubcoreMesh(**_MESH),
        scratch_types=[
            pltpu.VMEM((GPW // 128, 128), jnp.int32),
            pltpu.VMEM((128, D), jnp.float32),
            pltpu.SemaphoreType.DMA,
        ],
    )(_gather8_body)


_BN = 1000


def _vspec(bn=_BN, d=D):
    return pl.BlockSpec((bn, d), lambda i: (i, 0))


def _wspec():
    return pl.BlockSpec((D, D), lambda i: (0, 0))


def _bspec():
    return pl.BlockSpec((1, D), lambda i: (0, 0))


def _pre0_body(x_ref, z_ref, t_ref):
    xb = x_ref[...]
    t = jnp.mean(xb, axis=1, keepdims=True)
    t_ref[...] = t
    z_ref[...] = xb * jnp.tanh(t)


def _pre0(x):
    return pl.pallas_call(
        _pre0_body,
        grid=(N // _BN,),
        in_specs=[_vspec()],
        out_specs=[_vspec(), _vspec(_BN, 1)],
        out_shape=[jax.ShapeDtypeStruct((N, D), jnp.float32),
                   jax.ShapeDtypeStruct((N, 1), jnp.float32)],
    )(x)


def _combine_body(mode, p0_ref, p1_ref, cnt_ref, z_ref, wl_ref, wr_ref,
                  b_ref, *outs):
    den = jnp.maximum(cnt_ref[...], 1.0)
    mean = (p0_ref[...] + p1_ref[...]) / den
    h = (jnp.dot(mean, wl_ref[...], preferred_element_type=jnp.float32,
                 precision=lax.Precision.HIGHEST)
         + jnp.dot(z_ref[...], wr_ref[...], preferred_element_type=jnp.float32,
                   precision=lax.Precision.HIGHEST)
         + b_ref[...])
    if mode == "plain":
        outs[0][...] = h
        return
    y = jnp.maximum(h, 0.0)
    outs[0][...] = y
    if mode == "gate":
        t = jnp.mean(y, axis=1, keepdims=True)
        outs[2][...] = t
        outs[1][...] = y * jnp.tanh(t)


def _combine(mode, p0, p1, cnt, z, wl, wr, b):
    nout = {"gate": 3, "relu": 1, "plain": 1}[mode]
    out_specs = [_vspec(), _vspec(), _vspec(_BN, 1)][:nout]
    out_shape = [jax.ShapeDtypeStruct((N, D), jnp.float32),
                 jax.ShapeDtypeStruct((N, D), jnp.float32),
                 jax.ShapeDtypeStruct((N, 1), jnp.float32)][:nout]
    res = pl.pallas_call(
        functools.partial(_combine_body, mode),
        grid=(N // _BN,),
        in_specs=[_vspec(), _vspec(), _vspec(_BN, 1), _vspec(),
                  _wspec(), _wspec(), _bspec()],
        out_specs=out_specs,
        out_shape=out_shape,
    )(p0, p1, cnt, z, wl, wr, b.reshape(1, D))
    return res if nout > 1 else res[0]


def _add_body(a_ref, b_ref, o_ref):
    o_ref[...] = a_ref[...] + b_ref[...]


def _add(a, b):
    return pl.pallas_call(
        _add_body,
        grid=(N // _BN,),
        in_specs=[_vspec(), _vspec()],
        out_specs=_vspec(),
        out_shape=jax.ShapeDtypeStruct((N, D), jnp.float32),
    )(a, b)


def _loss_body(x_ref, y1_ref, a_ref, b_ref, c_ref, d_ref, f_ref, g_ref,
               h_ref, o_ref):
    i = pl.program_id(0)
    x, y1 = x_ref[...], y1_ref[...]
    A, B, C, Dv = a_ref[...], b_ref[...], c_ref[...], d_ref[...]
    F, G, H = f_ref[...], g_ref[...], h_ref[...]
    sq = lambda u, v: jnp.sum((u - v) ** 2)
    ab = lambda u, v: jnp.sum(jnp.abs(u - v))
    vals = jnp.stack([sq(A, B), sq(C, Dv), sq(y1, x), sq(C, G), sq(F, H),
                      ab(A, x), ab(C, A), ab(F, C)])

    @pl.when(i == 0)
    def _():
        o_ref[...] = jnp.zeros_like(o_ref)

    o_ref[...] += vals[None, :]


def _losses(x, y1, A, B, C, Dv, F, G, H):
    return pl.pallas_call(
        _loss_body,
        grid=(N // _BN,),
        in_specs=[_vspec()] * 9,
        out_specs=pl.BlockSpec((1, 8), lambda i: (0, 0)),
        out_shape=jax.ShapeDtypeStruct((1, 8), jnp.float32),
    )(x, y1, A, B, C, Dv, F, G, H)


def kernel(x, edge_index,
           W_l_0, W_r_0, b_0, W_l_1, W_r_1, b_1, W_l_2, W_r_2, b_2,
           W_l_3, W_r_3, b_3, W_l_4, W_r_4, b_4, W_l_5, W_r_5, b_5):
    E = edge_index.shape[1]
    per_tile = -(-E // (NW * 512)) * 512
    EP = per_tile * NW
    pad = EP - E
    ar = jnp.arange(pad, dtype=jnp.int32)
    srcp = jnp.concatenate([edge_index[0], ar % 32])
    dstp = jnp.concatenate([edge_index[1], N + (ar % 8)])
    zeros2 = jnp.zeros((N_PAD, D), jnp.float32)
    zeros1 = jnp.zeros((N_PAD,), jnp.float32)
    ones1 = jnp.ones((128,), jnp.float32)

    agg = _make_agg(EP)
    cnt_parts = _make_counts()(dstp, zeros1, ones1)
    cnt = (cnt_parts[:N_PAD] + cnt_parts[N_PAD:])[:N].reshape(N, 1)

    z0, t0 = _pre0(x)
    parts0 = agg(z0, srcp, dstp, zeros2)
    y1, z1, t1 = _combine("gate", parts0[:N], parts0[N_PAD:N_PAD + N], cnt,
                          z0, W_l_0, W_r_0, b_0)
    parts1 = agg(z1, srcp, dstp, zeros2)
    y2, z2, t2 = _combine("gate", parts1[:N], parts1[N_PAD:N_PAD + N], cnt,
                          z1, W_l_1, W_r_1, b_1)
    parts2 = agg(z2, srcp, dstp, zeros2)
    y3 = _combine("relu", parts2[:N], parts2[N_PAD:N_PAD + N], cnt,
                  z2, W_l_2, W_r_2, b_2)

    t0f, t1f, t2f = t0[:, 0], t1[:, 0], t2[:, 0]
    c1 = jnp.argsort(-t0f).astype(jnp.int32)
    p1 = jnp.argsort(-t1f[c1]).astype(jnp.int32)
    c2 = c1[p1]
    p2 = jnp.argsort(-t2f[c2]).astype(jnp.int32)
    c3 = c2[p2]
    inv0 = jnp.zeros((N,), jnp.int32).at[c1].set(
        jnp.arange(N, dtype=jnp.int32))
    u = c3[inv0]

    pad_i = (jnp.arange(NG - N, dtype=jnp.int32) % 32)

    def pidx(a):
        return jnp.concatenate([a, pad_i])

    A, B, C, Dv, F, G, H, Y3U = _make_gather8()(
        x, y1, z1, y2, z2, y3,
        pidx(c1), pidx(c2), pidx(c3), pidx(p1), pidx(p2), pidx(u))

    xu = _add(Y3U[:N], x)
    parts3 = agg(xu, srcp, dstp, zeros2)
    y4 = _combine("relu", parts3[:N], parts3[N_PAD:N_PAD + N], cnt,
                  xu, W_l_3, W_r_3, b_3)
    parts4 = agg(y4, srcp, dstp, zeros2)
    out = _combine("plain", parts4[:N], parts4[N_PAD:N_PAD + N], cnt,
                   y4, W_l_5, W_r_5, b_5)

    sums = _losses(x, y1, A[:N], B[:N], C[:N], Dv[:N], F[:N], G[:N], H[:N])
    l = sums[0] / jnp.float32(N * D)
    return (out, l[0], l[1], l[2], l[3], l[4], l[5], l[6], l[7])


# speedup vs baseline: 19.9313x; 1.3895x over previous
import functools

import jax
import jax.numpy as jnp
from jax import lax
from jax.experimental import pallas as pl
from jax.experimental.pallas import tpu as pltpu
from jax.experimental.pallas import tpu_sc as plsc

N = 10000
D = 128
NC, NS = 2, 16
NW = NC * NS
N_PAD = 10240
RPT = N_PAD // NS
NG = 12288
GPW = NG // NW

_MESH = dict(core_axis_name="c", subcore_axis_name="s", num_cores=NC,
             num_subcores=NS)


def _agg_body(with_cnt, *refs):
    if with_cnt:
        (zs, srcs, dsts, zeros2, zeros1, ones1, out, cntp,
         acc, cacc, srcb, dstb, rows2, onesb, gsA, gsB, ssA, ssB,
         csem) = refs
    else:
        (zs, srcs, dsts, zeros2, out,
         acc, srcb, dstb, rows2, gsA, gsB, ssA, ssB) = refs
    c = lax.axis_index("c")
    s = lax.axis_index("s")
    w = c * NS + s
    r0 = s * RPT
    pltpu.sync_copy(zeros2.at[pl.ds(r0, RPT)], acc.at[pl.ds(r0, RPT)])
    if with_cnt:
        pltpu.sync_copy(zeros1.at[pl.ds(r0, RPT)], cacc.at[pl.ds(r0, RPT)])
        pltpu.sync_copy(ones1, onesb)
    plsc.subcore_barrier()
    nrows = srcs.shape[0] // NW
    half = nrows // 2
    gs = (gsA, gsB)
    ss = (ssA, ssB)

    for h in range(2):
        pltpu.sync_copy(srcs.at[pl.ds(w * nrows + h * half, half)], srcb)
        pltpu.sync_copy(dsts.at[pl.ds(w * nrows + h * half, half)], dstb)

        def fire(j, b):
            g = pltpu.async_copy(zs.at[srcb.at[j]], rows2.at[b], gs[b])
            if with_cnt:
                pltpu.async_copy(onesb, cacc.at[dstb.at[j]], csem, add=True)
            return g

        def put(j, b, g):
            g.wait()
            pltpu.async_copy(rows2.at[b], acc.at[dstb.at[j]], ss[b],
                             add=True)

        def drain_s(b):
            pltpu.make_async_copy(zeros2.at[pl.ds(0, 128)], rows2.at[b],
                                  ss[b]).wait()

        g0 = fire(0, 0)
        g1 = fire(1, 1)
        put(0, 0, g0)
        put(1, 1, g1)

        def blk(i, carry):
            j = i * 2
            drain_s(0)
            ga = fire(j, 0)
            drain_s(1)
            gb = fire(j + 1, 1)
            put(j, 0, ga)
            put(j + 1, 1, gb)
            return carry

        lax.fori_loop(1, half // 2, blk, 0)
        drain_s(0)
        drain_s(1)

    if with_cnt:
        def cdr(i, carry):
            pltpu.make_async_copy(ones1, onesb, csem).wait()
            return carry

        lax.fori_loop(0, nrows, cdr, 0)
    plsc.subcore_barrier()
    pltpu.sync_copy(acc.at[pl.ds(r0, RPT)],
                    out.at[pl.ds(c * N_PAD + r0, RPT)])
    if with_cnt:
        pltpu.sync_copy(cacc.at[pl.ds(r0, RPT)],
                        cntp.at[pl.ds(c * N_PAD + r0, RPT)])


def _make_agg(with_cnt):
    half = 40
    out_type = [jax.ShapeDtypeStruct((NC * N_PAD, D), jnp.float32)]
    scratch = [
        pltpu.VMEM_SHARED((N_PAD, D), jnp.float32),
        pltpu.VMEM((half, 128), jnp.int32),
        pltpu.VMEM((half, 128), jnp.int32),
        pltpu.VMEM((2, 128, D), jnp.float32),
        pltpu.SemaphoreType.DMA,
        pltpu.SemaphoreType.DMA,
        pltpu.SemaphoreType.DMA,
        pltpu.SemaphoreType.DMA,
    ]
    if with_cnt:
        out_type.append(jax.ShapeDtypeStruct((NC * N_PAD,), jnp.float32))
        scratch = ([pltpu.VMEM_SHARED((N_PAD, D), jnp.float32),
                    pltpu.VMEM_SHARED((N_PAD,), jnp.float32)]
                   + scratch[1:4]
                   + [pltpu.VMEM((128,), jnp.float32)]
                   + scratch[4:]
                   + [pltpu.SemaphoreType.DMA])
    return functools.partial(
        pl.kernel,
        out_type=out_type if with_cnt else out_type[0],
        mesh=plsc.VectorSubcoreMesh(**_MESH),
        scratch_types=scratch,
    )(functools.partial(_agg_body, with_cnt))


def _gather8_body(xs, y1s, z1s, y2s, z2s, y3s, c1, c2, c3, p1, p2, uu,
                  oA, oB, oC, oD, oF, oG, oH, oU, idxb, rows, gsem, isem,
                  wsA, wsB):
    c = lax.axis_index("c")
    s = lax.axis_index("s")
    w = c * NS + s
    ncp = GPW // 128
    specs = [(y1s, c1, oA), (z1s, c2, oB), (y2s, c2, oC), (z2s, c3, oD),
             (y3s, c3, oF), (xs, p1, oG), (xs, p2, oH), (y3s, uu, oU)]
    stg = []
    for t, (src_ref, idx_ref, out_ref) in enumerate(specs):
        for ch in range(ncp):
            stg.append(pltpu.async_copy(
                idx_ref.at[pl.ds(w * GPW + ch * 128, 128)],
                idxb.at[t * ncp + ch], isem))
    for d in stg:
        d.wait()
    wdesc = [None, None]
    wsem = (wsA, wsB)
    k = 0
    for t, (src_ref, idx_ref, out_ref) in enumerate(specs):
        for ch in range(ncp):
            b = k % 2
            if wdesc[b] is not None:
                wdesc[b].wait()
            g = pltpu.async_copy(src_ref.at[idxb.at[t * ncp + ch]],
                                 rows.at[b], gsem)
            g.wait()
            wdesc[b] = pltpu.async_copy(
                rows.at[b], out_ref.at[pl.ds(w * GPW + ch * 128, 128)],
                wsem[b])
            k += 1
    wdesc[0].wait()
    wdesc[1].wait()


def _make_gather8():
    ot = [jax.ShapeDtypeStruct((NG, D), jnp.float32) for _ in range(8)]
    return functools.partial(
        pl.kernel,
        out_type=ot,
        mesh=plsc.VectorSubcoreMesh(**_MESH),
        scratch_types=[
            pltpu.VMEM((8 * (GPW // 128), 128), jnp.int32),
            pltpu.VMEM((2, 128, D), jnp.float32),
            pltpu.SemaphoreType.DMA,
            pltpu.SemaphoreType.DMA,
            pltpu.SemaphoreType.DMA,
            pltpu.SemaphoreType.DMA,
        ],
    )(_gather8_body)


_BN = 1000


def _vspec(bn=_BN, d=D):
    return pl.BlockSpec((bn, d), lambda i: (i, 0))


def _wspec():
    return pl.BlockSpec((D, D), lambda i: (0, 0))


def _bspec():
    return pl.BlockSpec((1, D), lambda i: (0, 0))


def _pre0_body(x_ref, z_ref, t_ref):
    xb = x_ref[...]
    t = jnp.mean(xb, axis=1, keepdims=True)
    t_ref[...] = t
    z_ref[...] = xb * jnp.tanh(t)


def _pre0(x):
    return pl.pallas_call(
        _pre0_body,
        grid=(N // _BN,),
        in_specs=[_vspec()],
        out_specs=[_vspec(), _vspec(_BN, 1)],
        out_shape=[jax.ShapeDtypeStruct((N, D), jnp.float32),
                   jax.ShapeDtypeStruct((N, 1), jnp.float32)],
    )(x)


def _combine_body(mode, p0_ref, p1_ref, cnt_ref, z_ref, wl_ref, wr_ref,
                  b_ref, *outs):
    den = jnp.maximum(cnt_ref[...], 1.0)
    mean = (p0_ref[...] + p1_ref[...]) / den
    h = (jnp.dot(mean, wl_ref[...], preferred_element_type=jnp.float32,
                 precision=lax.Precision.HIGHEST)
         + jnp.dot(z_ref[...], wr_ref[...], preferred_element_type=jnp.float32,
                   precision=lax.Precision.HIGHEST)
         + b_ref[...])
    if mode == "plain":
        outs[0][...] = h
        return
    y = jnp.maximum(h, 0.0)
    outs[0][...] = y
    if mode == "gate":
        t = jnp.mean(y, axis=1, keepdims=True)
        outs[2][...] = t
        outs[1][...] = y * jnp.tanh(t)


def _combine(mode, p0, p1, cnt, z, wl, wr, b):
    nout = {"gate": 3, "relu": 1, "plain": 1}[mode]
    out_specs = [_vspec(), _vspec(), _vspec(_BN, 1)][:nout]
    out_shape = [jax.ShapeDtypeStruct((N, D), jnp.float32),
                 jax.ShapeDtypeStruct((N, D), jnp.float32),
                 jax.ShapeDtypeStruct((N, 1), jnp.float32)][:nout]
    res = pl.pallas_call(
        functools.partial(_combine_body, mode),
        grid=(N // _BN,),
        in_specs=[_vspec(), _vspec(), _vspec(_BN, 1), _vspec(),
                  _wspec(), _wspec(), _bspec()],
        out_specs=out_specs,
        out_shape=out_shape,
    )(p0, p1, cnt, z, wl, wr, b.reshape(1, D))
    return res if nout > 1 else res[0]


def _add_body(a_ref, b_ref, o_ref):
    o_ref[...] = a_ref[...] + b_ref[...]


def _add(a, b):
    return pl.pallas_call(
        _add_body,
        grid=(N // _BN,),
        in_specs=[_vspec(), _vspec()],
        out_specs=_vspec(),
        out_shape=jax.ShapeDtypeStruct((N, D), jnp.float32),
    )(a, b)


def _loss_body(x_ref, y1_ref, a_ref, b_ref, c_ref, d_ref, f_ref, g_ref,
               h_ref, o_ref):
    i = pl.program_id(0)
    x, y1 = x_ref[...], y1_ref[...]
    A, B, C, Dv = a_ref[...], b_ref[...], c_ref[...], d_ref[...]
    F, G, H = f_ref[...], g_ref[...], h_ref[...]
    sq = lambda u, v: jnp.sum((u - v) ** 2)
    ab = lambda u, v: jnp.sum(jnp.abs(u - v))
    vals = jnp.stack([sq(A, B), sq(C, Dv), sq(y1, x), sq(C, G), sq(F, H),
                      ab(A, x), ab(C, A), ab(F, C)])

    @pl.when(i == 0)
    def _():
        o_ref[...] = jnp.zeros_like(o_ref)

    o_ref[...] += vals[None, :]


def _losses(x, y1, A, B, C, Dv, F, G, H):
    return pl.pallas_call(
        _loss_body,
        grid=(N // _BN,),
        in_specs=[_vspec()] * 9,
        out_specs=pl.BlockSpec((1, 8), lambda i: (0, 0)),
        out_shape=jax.ShapeDtypeStruct((1, 8), jnp.float32),
    )(x, y1, A, B, C, Dv, F, G, H)


def kernel(x, edge_index,
           W_l_0, W_r_0, b_0, W_l_1, W_r_1, b_1, W_l_2, W_r_2, b_2,
           W_l_3, W_r_3, b_3, W_l_4, W_r_4, b_4, W_l_5, W_r_5, b_5):
    E = edge_index.shape[1]
    per_tile = -(-E // (NW * 512)) * 512
    EP = per_tile * NW
    pad = EP - E
    ar = jnp.arange(pad, dtype=jnp.int32)
    srcp = jnp.concatenate([edge_index[0], ar % 32]).reshape(EP // 128, 128)
    dstp = jnp.concatenate([edge_index[1], N + (ar % 8)]).reshape(
        EP // 128, 128)
    zeros2 = jnp.zeros((N_PAD, D), jnp.float32)
    zeros1 = jnp.zeros((N_PAD,), jnp.float32)
    ones1 = jnp.ones((128,), jnp.float32)

    agg = _make_agg(False)

    z0, t0 = _pre0(x)
    parts0, cntp = _make_agg(True)(z0, srcp, dstp, zeros2, zeros1, ones1)
    cnt = (cntp[:N_PAD] + cntp[N_PAD:])[:N].reshape(N, 1)
    y1, z1, t1 = _combine("gate", parts0[:N], parts0[N_PAD:N_PAD + N], cnt,
                          z0, W_l_0, W_r_0, b_0)
    parts1 = agg(z1, srcp, dstp, zeros2)
    y2, z2, t2 = _combine("gate", parts1[:N], parts1[N_PAD:N_PAD + N], cnt,
                          z1, W_l_1, W_r_1, b_1)
    parts2 = agg(z2, srcp, dstp, zeros2)
    y3 = _combine("relu", parts2[:N], parts2[N_PAD:N_PAD + N], cnt,
                  z2, W_l_2, W_r_2, b_2)

    t0f, t1f, t2f = t0[:, 0], t1[:, 0], t2[:, 0]
    c1 = jnp.argsort(-t0f).astype(jnp.int32)
    p1 = jnp.argsort(-t1f[c1]).astype(jnp.int32)
    c2 = c1[p1]
    p2 = jnp.argsort(-t2f[c2]).astype(jnp.int32)
    c3 = c2[p2]
    inv0 = jnp.zeros((N,), jnp.int32).at[c1].set(
        jnp.arange(N, dtype=jnp.int32))
    u = c3[inv0]

    pad_i = (jnp.arange(NG - N, dtype=jnp.int32) % 32)

    def pidx(a):
        return jnp.concatenate([a, pad_i])

    A, B, C, Dv, F, G, H, Y3U = _make_gather8()(
        x, y1, z1, y2, z2, y3,
        pidx(c1), pidx(c2), pidx(c3), pidx(p1), pidx(p2), pidx(u))

    xu = _add(Y3U[:N], x)
    parts3 = agg(xu, srcp, dstp, zeros2)
    y4 = _combine("relu", parts3[:N], parts3[N_PAD:N_PAD + N], cnt,
                  xu, W_l_3, W_r_3, b_3)
    parts4 = agg(y4, srcp, dstp, zeros2)
    out = _combine("plain", parts4[:N], parts4[N_PAD:N_PAD + N], cnt,
                   y4, W_l_5, W_r_5, b_5)

    sums = _losses(x, y1, A[:N], B[:N], C[:N], Dv[:N], F[:N], G[:N], H[:N])
    l = sums[0] / jnp.float32(N * D)
    return (out, l[0], l[1], l[2], l[3], l[4], l[5], l[6], l[7])


# speedup vs baseline: 20.0515x; 1.0060x over previous
import functools

import jax
import jax.numpy as jnp
from jax import lax
from jax.experimental import pallas as pl
from jax.experimental.pallas import tpu as pltpu
from jax.experimental.pallas import tpu_sc as plsc

N = 10000
D = 128
NC, NS = 2, 16
NW = NC * NS
N_PAD = 10240
RPT = N_PAD // NS
NG = 12288
GPW = NG // NW

_MESH = dict(core_axis_name="c", subcore_axis_name="s", num_cores=NC,
             num_subcores=NS)


def _agg_body(with_cnt, *refs):
    if with_cnt:
        (zs, srcs, dsts, zeros2, zeros1, ones1, out, cntp,
         acc, cacc, srcb, dstb, rows2, onesb, gsA, gsB, ssA, ssB,
         csem) = refs
    else:
        (zs, srcs, dsts, zeros2, out,
         acc, srcb, dstb, rows2, gsA, gsB, ssA, ssB) = refs
    c = lax.axis_index("c")
    s = lax.axis_index("s")
    w = c * NS + s
    r0 = s * RPT
    pltpu.sync_copy(zeros2.at[pl.ds(r0, RPT)], acc.at[pl.ds(r0, RPT)])
    if with_cnt:
        pltpu.sync_copy(zeros1.at[pl.ds(r0, RPT)], cacc.at[pl.ds(r0, RPT)])
        pltpu.sync_copy(ones1, onesb)
    plsc.subcore_barrier()
    nrows = srcs.shape[0] // NW
    half = nrows // 2
    gs = (gsA, gsB)
    ss = (ssA, ssB)

    for h in range(2):
        pltpu.sync_copy(srcs.at[pl.ds(w * nrows + h * half, half)], srcb)
        pltpu.sync_copy(dsts.at[pl.ds(w * nrows + h * half, half)], dstb)

        def fire(j, b):
            g = pltpu.async_copy(zs.at[srcb.at[j]], rows2.at[b], gs[b])
            if with_cnt:
                pltpu.async_copy(onesb, cacc.at[dstb.at[j]], csem, add=True)
            return g

        def put(j, b, g):
            g.wait()
            pltpu.async_copy(rows2.at[b], acc.at[dstb.at[j]], ss[b],
                             add=True)

        def drain_s(b):
            pltpu.make_async_copy(zeros2.at[pl.ds(0, 128)], rows2.at[b],
                                  ss[b]).wait()

        g0 = fire(0, 0)
        g1 = fire(1, 1)
        put(0, 0, g0)
        put(1, 1, g1)

        def blk(i, carry):
            j = i * 2
            drain_s(0)
            ga = fire(j, 0)
            drain_s(1)
            gb = fire(j + 1, 1)
            put(j, 0, ga)
            put(j + 1, 1, gb)
            return carry

        lax.fori_loop(1, half // 2, blk, 0)
        drain_s(0)
        drain_s(1)

    if with_cnt:
        def cdr(i, carry):
            pltpu.make_async_copy(ones1, onesb, csem).wait()
            return carry

        lax.fori_loop(0, nrows, cdr, 0)
    plsc.subcore_barrier()
    pltpu.sync_copy(acc.at[pl.ds(r0, RPT)],
                    out.at[pl.ds(c * N_PAD + r0, RPT)])
    if with_cnt:
        pltpu.sync_copy(cacc.at[pl.ds(r0, RPT)],
                        cntp.at[pl.ds(c * N_PAD + r0, RPT)])


def _make_agg(with_cnt):
    half = 40
    out_type = [jax.ShapeDtypeStruct((NC * N_PAD, D), jnp.float32)]
    scratch = [
        pltpu.VMEM_SHARED((N_PAD, D), jnp.float32),
        pltpu.VMEM((half, 128), jnp.int32),
        pltpu.VMEM((half, 128), jnp.int32),
        pltpu.VMEM((2, 128, D), jnp.float32),
        pltpu.SemaphoreType.DMA,
        pltpu.SemaphoreType.DMA,
        pltpu.SemaphoreType.DMA,
        pltpu.SemaphoreType.DMA,
    ]
    if with_cnt:
        out_type.append(jax.ShapeDtypeStruct((NC * N_PAD,), jnp.float32))
        scratch = ([pltpu.VMEM_SHARED((N_PAD, D), jnp.float32),
                    pltpu.VMEM_SHARED((N_PAD,), jnp.float32)]
                   + scratch[1:4]
                   + [pltpu.VMEM((128,), jnp.float32)]
                   + scratch[4:]
                   + [pltpu.SemaphoreType.DMA])
    return functools.partial(
        pl.kernel,
        out_type=out_type if with_cnt else out_type[0],
        mesh=plsc.VectorSubcoreMesh(**_MESH),
        scratch_types=scratch,
    )(functools.partial(_agg_body, with_cnt))


def _gather8_body(xs, y1s, z1s, y2s, z2s, y3s, c2v, c1g, c1s, c2p, p1, p2,
                  oA, oB, oC, oD, oF, oG, oH, oU, idxb, rows, gsem, isem,
                  wsA, wsB):
    c = lax.axis_index("c")
    s = lax.axis_index("s")
    w = c * NS + s
    ncp = GPW // 128
    stg_idx = [c1g, c1s, c2p, p1, p2]
    stg = []
    for t, idx_ref in enumerate(stg_idx):
        for ch in range(ncp):
            stg.append(pltpu.async_copy(
                idx_ref.at[pl.ds(w * GPW + ch * 128, 128)],
                idxb.at[t * ncp + ch], isem))
    for d in stg:
        d.wait()
    for ch in range(ncp):
        pltpu.async_copy(c2v.at[idxb.at[4 * ncp + ch]],
                         idxb.at[5 * ncp + ch], isem).wait()
    specs = [(y1s, 0, oA), (z1s, 2, oB), (y2s, 2, oC), (z2s, 5, oD),
             (y3s, 5, oF), (xs, 3, oG), (xs, 4, oH)]
    wdesc = [[], []]
    wsem = (wsA, wsB)
    k = 0
    for src_ref, t, out_ref in specs:
        for ch in range(ncp):
            b = k % 2
            for d in wdesc[b]:
                d.wait()
            wdesc[b] = []
            g = pltpu.async_copy(src_ref.at[idxb.at[t * ncp + ch]],
                                 rows.at[b], gsem)
            g.wait()
            wdesc[b].append(pltpu.async_copy(
                rows.at[b], out_ref.at[pl.ds(w * GPW + ch * 128, 128)],
                wsem[b]))
            if out_ref is oF:
                wdesc[b].append(pltpu.async_copy(
                    rows.at[b], oU.at[idxb.at[1 * ncp + ch]], wsem[b]))
            k += 1
    for b in range(2):
        for d in wdesc[b]:
            d.wait()


def _make_gather8():
    ot = [jax.ShapeDtypeStruct((NG, D), jnp.float32) for _ in range(7)]
    ot.append(jax.ShapeDtypeStruct((N_PAD, D), jnp.float32))
    return functools.partial(
        pl.kernel,
        out_type=ot,
        mesh=plsc.VectorSubcoreMesh(**_MESH),
        scratch_types=[
            pltpu.VMEM((6 * (GPW // 128), 128), jnp.int32),
            pltpu.VMEM((2, 128, D), jnp.float32),
            pltpu.SemaphoreType.DMA,
            pltpu.SemaphoreType.DMA,
            pltpu.SemaphoreType.DMA,
            pltpu.SemaphoreType.DMA,
        ],
    )(_gather8_body)


_BN = 1000


def _vspec(bn=_BN, d=D):
    return pl.BlockSpec((bn, d), lambda i: (i, 0))


def _wspec():
    return pl.BlockSpec((D, D), lambda i: (0, 0))


def _bspec():
    return pl.BlockSpec((1, D), lambda i: (0, 0))


def _pre0_body(x_ref, z_ref, t_ref):
    xb = x_ref[...]
    t = jnp.mean(xb, axis=1, keepdims=True)
    t_ref[...] = t
    z_ref[...] = xb * jnp.tanh(t)


def _pre0(x):
    return pl.pallas_call(
        _pre0_body,
        grid=(N // _BN,),
        in_specs=[_vspec()],
        out_specs=[_vspec(), _vspec(_BN, 1)],
        out_shape=[jax.ShapeDtypeStruct((N, D), jnp.float32),
                   jax.ShapeDtypeStruct((N, 1), jnp.float32)],
    )(x)


def _combine_body(mode, p0_ref, p1_ref, cnt_ref, z_ref, wl_ref, wr_ref,
                  b_ref, *outs):
    den = jnp.maximum(cnt_ref[...], 1.0)
    mean = (p0_ref[...] + p1_ref[...]) / den
    h = (jnp.dot(mean, wl_ref[...], preferred_element_type=jnp.float32,
                 precision=lax.Precision.HIGHEST)
         + jnp.dot(z_ref[...], wr_ref[...], preferred_element_type=jnp.float32,
                   precision=lax.Precision.HIGHEST)
         + b_ref[...])
    if mode == "plain":
        outs[0][...] = h
        return
    y = jnp.maximum(h, 0.0)
    outs[0][...] = y
    if mode == "gate":
        t = jnp.mean(y, axis=1, keepdims=True)
        outs[2][...] = t
        outs[1][...] = y * jnp.tanh(t)


def _combine(mode, p0, p1, cnt, z, wl, wr, b):
    nout = {"gate": 3, "relu": 1, "plain": 1}[mode]
    out_specs = [_vspec(), _vspec(), _vspec(_BN, 1)][:nout]
    out_shape = [jax.ShapeDtypeStruct((N, D), jnp.float32),
                 jax.ShapeDtypeStruct((N, D), jnp.float32),
                 jax.ShapeDtypeStruct((N, 1), jnp.float32)][:nout]
    res = pl.pallas_call(
        functools.partial(_combine_body, mode),
        grid=(N // _BN,),
        in_specs=[_vspec(), _vspec(), _vspec(_BN, 1), _vspec(),
                  _wspec(), _wspec(), _bspec()],
        out_specs=out_specs,
        out_shape=out_shape,
    )(p0, p1, cnt, z, wl, wr, b.reshape(1, D))
    return res if nout > 1 else res[0]


def _add_body(a_ref, b_ref, o_ref):
    o_ref[...] = a_ref[...] + b_ref[...]


def _add(a, b):
    return pl.pallas_call(
        _add_body,
        grid=(N // _BN,),
        in_specs=[_vspec(), _vspec()],
        out_specs=_vspec(),
        out_shape=jax.ShapeDtypeStruct((N, D), jnp.float32),
    )(a, b)


def _loss_body(x_ref, y1_ref, a_ref, b_ref, c_ref, d_ref, f_ref, g_ref,
               h_ref, o_ref):
    i = pl.program_id(0)
    x, y1 = x_ref[...], y1_ref[...]
    A, B, C, Dv = a_ref[...], b_ref[...], c_ref[...], d_ref[...]
    F, G, H = f_ref[...], g_ref[...], h_ref[...]
    sq = lambda u, v: jnp.sum((u - v) ** 2)
    ab = lambda u, v: jnp.sum(jnp.abs(u - v))
    vals = jnp.stack([sq(A, B), sq(C, Dv), sq(y1, x), sq(C, G), sq(F, H),
                      ab(A, x), ab(C, A), ab(F, C)])

    @pl.when(i == 0)
    def _():
        o_ref[...] = jnp.zeros_like(o_ref)

    o_ref[...] += vals[None, :]


def _losses(x, y1, A, B, C, Dv, F, G, H):
    return pl.pallas_call(
        _loss_body,
        grid=(N // _BN,),
        in_specs=[_vspec()] * 9,
        out_specs=pl.BlockSpec((1, 8), lambda i: (0, 0)),
        out_shape=jax.ShapeDtypeStruct((1, 8), jnp.float32),
    )(x, y1, A, B, C, Dv, F, G, H)


def kernel(x, edge_index,
           W_l_0, W_r_0, b_0, W_l_1, W_r_1, b_1, W_l_2, W_r_2, b_2,
           W_l_3, W_r_3, b_3, W_l_4, W_r_4, b_4, W_l_5, W_r_5, b_5):
    E = edge_index.shape[1]
    per_tile = -(-E // (NW * 512)) * 512
    EP = per_tile * NW
    pad = EP - E
    ar = jnp.arange(pad, dtype=jnp.int32)
    srcp = jnp.concatenate([edge_index[0], ar % 32]).reshape(EP // 128, 128)
    dstp = jnp.concatenate([edge_index[1], N + (ar % 8)]).reshape(
        EP // 128, 128)
    zeros2 = jnp.zeros((N_PAD, D), jnp.float32)
    zeros1 = jnp.zeros((N_PAD,), jnp.float32)
    ones1 = jnp.ones((128,), jnp.float32)

    agg = _make_agg(False)

    z0, t0 = _pre0(x)
    sums0, cntp = _make_agg(True)(z0, srcp, dstp, zeros2, zeros1, ones1)
    cnt = (cntp[:N_PAD] + cntp[N_PAD:])[:N].reshape(N, 1)
    y1, z1, t1 = _combine("gate", sums0[:N], sums0[N_PAD:N_PAD + N], cnt,
                          z0, W_l_0, W_r_0, b_0)
    sums1 = agg(z1, srcp, dstp, zeros2)
    y2, z2, t2 = _combine("gate", sums1[:N], sums1[N_PAD:N_PAD + N], cnt,
                          z1, W_l_1, W_r_1, b_1)
    sums2 = agg(z2, srcp, dstp, zeros2)
    y3 = _combine("relu", sums2[:N], sums2[N_PAD:N_PAD + N], cnt,
                  z2, W_l_2, W_r_2, b_2)

    t0f, t1f, t2f = t0[:, 0], t1[:, 0], t2[:, 0]
    c1 = jnp.argsort(-t0f).astype(jnp.int32)
    p1 = jnp.argsort(-t1f[c1]).astype(jnp.int32)
    c2 = c1[p1]
    p2 = jnp.argsort(-t2f[c2]).astype(jnp.int32)

    npad = NG - N
    pad_g = jnp.arange(npad, dtype=jnp.int32) % 32
    pad_s = N + (jnp.arange(npad, dtype=jnp.int32) % 8)

    def pidx(a, pad):
        return jnp.concatenate([a, pad])

    A, B, C, Dv, F, G, H, Y3U = _make_gather8()(
        x, y1, z1, y2, z2, y3, c2,
        pidx(c1, pad_g), pidx(c1, pad_s), pidx(c2, pad_g),
        pidx(p1, pad_g), pidx(p2, pad_g))

    xu = _add(Y3U[:N], x)
    sums3 = agg(xu, srcp, dstp, zeros2)
    y4 = _combine("relu", sums3[:N], sums3[N_PAD:N_PAD + N], cnt,
                  xu, W_l_3, W_r_3, b_3)
    sums4 = agg(y4, srcp, dstp, zeros2)
    out = _combine("plain", sums4[:N], sums4[N_PAD:N_PAD + N], cnt,
                   y4, W_l_5, W_r_5, b_5)

    sums = _losses(x, y1, A[:N], B[:N], C[:N], Dv[:N], F[:N], G[:N], H[:N])
    l = sums[0] / jnp.float32(N * D)
    return (out, l[0], l[1], l[2], l[3], l[4], l[5], l[6], l[7])


# speedup vs baseline: 20.6478x; 1.0297x over previous
import functools

import jax
import jax.numpy as jnp
from jax import lax
from jax.experimental import pallas as pl
from jax.experimental.pallas import tpu as pltpu
from jax.experimental.pallas import tpu_sc as plsc

N = 10000
D = 128
NC, NS = 2, 16
NW = NC * NS
N_PAD = 10240
RPT = N_PAD // NS
NG = 12288
GPW = NG // NW

_MESH = dict(core_axis_name="c", subcore_axis_name="s", num_cores=NC,
             num_subcores=NS)


def _agg_body(with_cnt, *refs):
    if with_cnt:
        (zs, srcs, dsts, zeros2, zeros1, ones1, out, cntp,
         acc, cacc, srcb, dstb, rows2, onesb, gsA, gsB, ssA, ssB,
         csem) = refs
    else:
        (zs, srcs, dsts, zeros2, out,
         acc, srcb, dstb, rows2, gsA, gsB, ssA, ssB) = refs
    c = lax.axis_index("c")
    s = lax.axis_index("s")
    w = c * NS + s
    r0 = s * RPT
    pltpu.sync_copy(zeros2.at[pl.ds(r0, RPT)], acc.at[pl.ds(r0, RPT)])
    if with_cnt:
        pltpu.sync_copy(zeros1.at[pl.ds(r0, RPT)], cacc.at[pl.ds(r0, RPT)])
        pltpu.sync_copy(ones1, onesb)
    plsc.subcore_barrier()
    nrows = srcs.shape[0] // NW
    half = nrows // 2
    gs = (gsA, gsB)
    ss = (ssA, ssB)

    for h in range(2):
        pltpu.sync_copy(srcs.at[pl.ds(w * nrows + h * half, half)], srcb)
        pltpu.sync_copy(dsts.at[pl.ds(w * nrows + h * half, half)], dstb)

        def fire(j, b):
            g = pltpu.async_copy(zs.at[srcb.at[j]], rows2.at[b], gs[b])
            if with_cnt:
                pltpu.async_copy(onesb, cacc.at[dstb.at[j]], csem, add=True)
            return g

        def put(j, b, g):
            g.wait()
            pltpu.async_copy(rows2.at[b], acc.at[dstb.at[j]], ss[b],
                             add=True)

        def drain_s(b):
            pltpu.make_async_copy(zeros2.at[pl.ds(0, 128)], rows2.at[b],
                                  ss[b]).wait()

        g0 = fire(0, 0)
        g1 = fire(1, 1)
        put(0, 0, g0)
        put(1, 1, g1)

        def blk(i, carry):
            j = i * 2
            drain_s(0)
            ga = fire(j, 0)
            drain_s(1)
            gb = fire(j + 1, 1)
            put(j, 0, ga)
            put(j + 1, 1, gb)
            return carry

        lax.fori_loop(1, half // 2, blk, 0)
        drain_s(0)
        drain_s(1)

    if with_cnt:
        def cdr(i, carry):
            pltpu.make_async_copy(ones1, onesb, csem).wait()
            return carry

        lax.fori_loop(0, nrows, cdr, 0)
    plsc.subcore_barrier()
    pltpu.sync_copy(acc.at[pl.ds(r0, RPT)],
                    out.at[pl.ds(c * N_PAD + r0, RPT)])
    if with_cnt:
        pltpu.sync_copy(cacc.at[pl.ds(r0, RPT)],
                        cntp.at[pl.ds(c * N_PAD + r0, RPT)])


def _make_agg(with_cnt):
    half = 40
    out_type = [jax.ShapeDtypeStruct((NC * N_PAD, D), jnp.float32)]
    scratch = [
        pltpu.VMEM_SHARED((N_PAD, D), jnp.float32),
        pltpu.VMEM((half, 128), jnp.int32),
        pltpu.VMEM((half, 128), jnp.int32),
        pltpu.VMEM((2, 128, D), jnp.float32),
        pltpu.SemaphoreType.DMA,
        pltpu.SemaphoreType.DMA,
        pltpu.SemaphoreType.DMA,
        pltpu.SemaphoreType.DMA,
    ]
    if with_cnt:
        out_type.append(jax.ShapeDtypeStruct((NC * N_PAD,), jnp.float32))
        scratch = ([pltpu.VMEM_SHARED((N_PAD, D), jnp.float32),
                    pltpu.VMEM_SHARED((N_PAD,), jnp.float32)]
                   + scratch[1:4]
                   + [pltpu.VMEM((128,), jnp.float32)]
                   + scratch[4:]
                   + [pltpu.SemaphoreType.DMA])
    return functools.partial(
        pl.kernel,
        out_type=out_type if with_cnt else out_type[0],
        mesh=plsc.VectorSubcoreMesh(**_MESH),
        scratch_types=scratch,
    )(functools.partial(_agg_body, with_cnt))


def _gather8_body(xs, y1s, z1s, y2s, z2s, y3s, inv0v, invc2v,
                  c1g, c1s, c2p, c3p,
                  oA, oB, oC, oD, oF, oG, oH, oU, idxb, rows, gsem, isem,
                  wsA, wsB):
    c = lax.axis_index("c")
    s = lax.axis_index("s")
    w = c * NS + s
    ncp = GPW // 128
    stg_idx = [c1g, c1s, c2p, c3p]
    stg = []
    for t, idx_ref in enumerate(stg_idx):
        for ch in range(ncp):
            stg.append(pltpu.async_copy(
                idx_ref.at[pl.ds(w * GPW + ch * 128, 128)],
                idxb.at[t * ncp + ch], isem))
    for d in stg:
        d.wait()
    der = []
    for ch in range(ncp):
        der.append(pltpu.async_copy(inv0v.at[idxb.at[2 * ncp + ch]],
                                    idxb.at[4 * ncp + ch], isem))
    for ch in range(ncp):
        der.append(pltpu.async_copy(invc2v.at[idxb.at[3 * ncp + ch]],
                                    idxb.at[5 * ncp + ch], isem))
    for d in der:
        d.wait()
    specs = [(y1s, 0, oA), (z1s, 2, oB), (y2s, 2, oC), (z2s, 3, oD),
             (y3s, 3, oF), (xs, 4, oG), (xs, 5, oH)]
    wdesc = [[], []]
    wsem = (wsA, wsB)
    k = 0
    for src_ref, t, out_ref in specs:
        for ch in range(ncp):
            b = k % 2
            for d in wdesc[b]:
                d.wait()
            wdesc[b] = []
            g = pltpu.async_copy(src_ref.at[idxb.at[t * ncp + ch]],
                                 rows.at[b], gsem)
            g.wait()
            wdesc[b].append(pltpu.async_copy(
                rows.at[b], out_ref.at[pl.ds(w * GPW + ch * 128, 128)],
                wsem[b]))
            if out_ref is oF:
                wdesc[b].append(pltpu.async_copy(
                    rows.at[b], oU.at[idxb.at[1 * ncp + ch]], wsem[b]))
            k += 1
    for b in range(2):
        for d in wdesc[b]:
            d.wait()


def _make_gather8():
    ot = [jax.ShapeDtypeStruct((NG, D), jnp.float32) for _ in range(7)]
    ot.append(jax.ShapeDtypeStruct((N_PAD, D), jnp.float32))
    return functools.partial(
        pl.kernel,
        out_type=ot,
        mesh=plsc.VectorSubcoreMesh(**_MESH),
        scratch_types=[
            pltpu.VMEM((6 * (GPW // 128), 128), jnp.int32),
            pltpu.VMEM((2, 128, D), jnp.float32),
            pltpu.SemaphoreType.DMA,
            pltpu.SemaphoreType.DMA,
            pltpu.SemaphoreType.DMA,
            pltpu.SemaphoreType.DMA,
        ],
    )(_gather8_body)


_BN = 1000


def _vspec(bn=_BN, d=D):
    return pl.BlockSpec((bn, d), lambda i: (i, 0))


def _wspec():
    return pl.BlockSpec((D, D), lambda i: (0, 0))


def _bspec():
    return pl.BlockSpec((1, D), lambda i: (0, 0))


def _pre0_body(x_ref, z_ref, t_ref):
    xb = x_ref[...]
    t = jnp.mean(xb, axis=1, keepdims=True)
    t_ref[...] = t
    z_ref[...] = xb * jnp.tanh(t)


def _pre0(x):
    return pl.pallas_call(
        _pre0_body,
        grid=(N // _BN,),
        in_specs=[_vspec()],
        out_specs=[_vspec(), _vspec(_BN, 1)],
        out_shape=[jax.ShapeDtypeStruct((N, D), jnp.float32),
                   jax.ShapeDtypeStruct((N, 1), jnp.float32)],
    )(x)


def _combine_body(mode, p0_ref, p1_ref, cnt_ref, z_ref, wl_ref, wr_ref,
                  b_ref, *outs):
    den = jnp.maximum(cnt_ref[...], 1.0)
    mean = (p0_ref[...] + p1_ref[...]) / den
    h = (jnp.dot(mean, wl_ref[...], preferred_element_type=jnp.float32,
                 precision=lax.Precision.HIGHEST)
         + jnp.dot(z_ref[...], wr_ref[...], preferred_element_type=jnp.float32,
                   precision=lax.Precision.HIGHEST)
         + b_ref[...])
    if mode == "plain":
        outs[0][...] = h
        return
    y = jnp.maximum(h, 0.0)
    outs[0][...] = y
    if mode == "gate":
        t = jnp.mean(y, axis=1, keepdims=True)
        outs[2][...] = t
        outs[1][...] = y * jnp.tanh(t)


def _combine(mode, p0, p1, cnt, z, wl, wr, b):
    nout = {"gate": 3, "relu": 1, "plain": 1}[mode]
    out_specs = [_vspec(), _vspec(), _vspec(_BN, 1)][:nout]
    out_shape = [jax.ShapeDtypeStruct((N, D), jnp.float32),
                 jax.ShapeDtypeStruct((N, D), jnp.float32),
                 jax.ShapeDtypeStruct((N, 1), jnp.float32)][:nout]
    res = pl.pallas_call(
        functools.partial(_combine_body, mode),
        grid=(N // _BN,),
        in_specs=[_vspec(), _vspec(), _vspec(_BN, 1), _vspec(),
                  _wspec(), _wspec(), _bspec()],
        out_specs=out_specs,
        out_shape=out_shape,
    )(p0, p1, cnt, z, wl, wr, b.reshape(1, D))
    return res if nout > 1 else res[0]


def _add_body(a_ref, b_ref, o_ref):
    o_ref[...] = a_ref[...] + b_ref[...]


def _add(a, b):
    return pl.pallas_call(
        _add_body,
        grid=(N // _BN,),
        in_specs=[_vspec(), _vspec()],
        out_specs=_vspec(),
        out_shape=jax.ShapeDtypeStruct((N, D), jnp.float32),
    )(a, b)


def _loss_body(x_ref, y1_ref, a_ref, b_ref, c_ref, d_ref, f_ref, g_ref,
               h_ref, o_ref):
    i = pl.program_id(0)
    x, y1 = x_ref[...], y1_ref[...]
    A, B, C, Dv = a_ref[...], b_ref[...], c_ref[...], d_ref[...]
    F, G, H = f_ref[...], g_ref[...], h_ref[...]
    sq = lambda u, v: jnp.sum((u - v) ** 2)
    ab = lambda u, v: jnp.sum(jnp.abs(u - v))
    vals = jnp.stack([sq(A, B), sq(C, Dv), sq(y1, x), sq(C, G), sq(F, H),
                      ab(A, x), ab(C, A), ab(F, C)])

    @pl.when(i == 0)
    def _():
        o_ref[...] = jnp.zeros_like(o_ref)

    o_ref[...] += vals[None, :]


def _losses(x, y1, A, B, C, Dv, F, G, H):
    return pl.pallas_call(
        _loss_body,
        grid=(N // _BN,),
        in_specs=[_vspec()] * 9,
        out_specs=pl.BlockSpec((1, 8), lambda i: (0, 0)),
        out_shape=jax.ShapeDtypeStruct((1, 8), jnp.float32),
    )(x, y1, A, B, C, Dv, F, G, H)


def kernel(x, edge_index,
           W_l_0, W_r_0, b_0, W_l_1, W_r_1, b_1, W_l_2, W_r_2, b_2,
           W_l_3, W_r_3, b_3, W_l_4, W_r_4, b_4, W_l_5, W_r_5, b_5):
    E = edge_index.shape[1]
    per_tile = -(-E // (NW * 512)) * 512
    EP = per_tile * NW
    pad = EP - E
    ar = jnp.arange(pad, dtype=jnp.int32)
    srcp = jnp.concatenate([edge_index[0], ar % 32]).reshape(EP // 128, 128)
    dstp = jnp.concatenate([edge_index[1], N + (ar % 8)]).reshape(
        EP // 128, 128)
    zeros2 = jnp.zeros((N_PAD, D), jnp.float32)
    zeros1 = jnp.zeros((N_PAD,), jnp.float32)
    ones1 = jnp.ones((128,), jnp.float32)

    agg = _make_agg(False)

    z0, t0 = _pre0(x)
    sums0, cntp = _make_agg(True)(z0, srcp, dstp, zeros2, zeros1, ones1)
    cnt = (cntp[:N_PAD] + cntp[N_PAD:])[:N].reshape(N, 1)
    y1, z1, t1 = _combine("gate", sums0[:N], sums0[N_PAD:N_PAD + N], cnt,
                          z0, W_l_0, W_r_0, b_0)
    sums1 = agg(z1, srcp, dstp, zeros2)
    y2, z2, t2 = _combine("gate", sums1[:N], sums1[N_PAD:N_PAD + N], cnt,
                          z1, W_l_1, W_r_1, b_1)
    sums2 = agg(z2, srcp, dstp, zeros2)
    y3 = _combine("relu", sums2[:N], sums2[N_PAD:N_PAD + N], cnt,
                  z2, W_l_2, W_r_2, b_2)

    t0f, t1f, t2f = t0[:, 0], t1[:, 0], t2[:, 0]
    iot = jnp.arange(N, dtype=jnp.int32)
    _, c1 = lax.sort((-t0f, iot), num_keys=1, is_stable=True)
    _, inv0 = lax.sort((c1, iot), num_keys=1, is_stable=True)
    _, _, c2 = lax.sort((-t1f, inv0, iot), num_keys=2, is_stable=True)
    _, invc2 = lax.sort((c2, iot), num_keys=1, is_stable=True)
    _, _, c3 = lax.sort((-t2f, invc2, iot), num_keys=2, is_stable=True)

    npad = NG - N
    pad_g = jnp.arange(npad, dtype=jnp.int32) % 32
    pad_s = N + (jnp.arange(npad, dtype=jnp.int32) % 8)

    def pidx(a, pad):
        return jnp.concatenate([a, pad])

    A, B, C, Dv, F, G, H, Y3U = _make_gather8()(
        x, y1, z1, y2, z2, y3, inv0, invc2,
        pidx(c1, pad_g), pidx(c1, pad_s), pidx(c2, pad_g),
        pidx(c3, pad_g))

    xu = _add(Y3U[:N], x)
    sums3 = agg(xu, srcp, dstp, zeros2)
    y4 = _combine("relu", sums3[:N], sums3[N_PAD:N_PAD + N], cnt,
                  xu, W_l_3, W_r_3, b_3)
    sums4 = agg(y4, srcp, dstp, zeros2)
    out = _combine("plain", sums4[:N], sums4[N_PAD:N_PAD + N], cnt,
                   y4, W_l_5, W_r_5, b_5)

    sums = _losses(x, y1, A[:N], B[:N], C[:N], Dv[:N], F[:N], G[:N], H[:N])
    l = sums[0] / jnp.float32(N * D)
    return (out, l[0], l[1], l[2], l[3], l[4], l[5], l[6], l[7])


# speedup vs baseline: 21.5789x; 1.0451x over previous
import functools

import jax
import jax.numpy as jnp
from jax import lax
from jax.experimental import pallas as pl
from jax.experimental.pallas import tpu as pltpu
from jax.experimental.pallas import tpu_sc as plsc

N = 10000
D = 128
NC, NS = 2, 16
NW = NC * NS
N_PAD = 10240
RPT = N_PAD // NS
NG = 12288
GPW = NG // NW

_MESH = dict(core_axis_name="c", subcore_axis_name="s", num_cores=NC,
             num_subcores=NS)


def _agg_body(with_cnt, *refs):
    if with_cnt:
        (zs, srcs, dsts, zeros2, zeros1, ones1, out, cntp,
         acc, cacc, srcb, dstb, rows2, onesb, gsA, gsB, ssA, ssB,
         csem) = refs
    else:
        (zs, srcs, dsts, zeros2, out,
         acc, srcb, dstb, rows2, gsA, gsB, ssA, ssB) = refs
    c = lax.axis_index("c")
    s = lax.axis_index("s")
    w = c * NS + s
    r0 = s * RPT
    pltpu.sync_copy(zeros2.at[pl.ds(r0, RPT)], acc.at[pl.ds(r0, RPT)])
    if with_cnt:
        pltpu.sync_copy(zeros1.at[pl.ds(r0, RPT)], cacc.at[pl.ds(r0, RPT)])
        pltpu.sync_copy(ones1, onesb)
    plsc.subcore_barrier()
    nrows = srcs.shape[0] // NW
    half = nrows // 2
    gs = (gsA, gsB)
    ss = (ssA, ssB)

    for h in range(2):
        pltpu.sync_copy(srcs.at[pl.ds(w * nrows + h * half, half)], srcb)
        pltpu.sync_copy(dsts.at[pl.ds(w * nrows + h * half, half)], dstb)

        def fire(j, b):
            g = pltpu.async_copy(zs.at[srcb.at[j]], rows2.at[b], gs[b])
            if with_cnt:
                pltpu.async_copy(onesb, cacc.at[dstb.at[j]], csem, add=True)
            return g

        def put(j, b, g):
            g.wait()
            pltpu.async_copy(rows2.at[b], acc.at[dstb.at[j]], ss[b],
                             add=True)

        def drain_s(b):
            pltpu.make_async_copy(zeros2.at[pl.ds(0, 128)], rows2.at[b],
                                  ss[b]).wait()

        g0 = fire(0, 0)
        g1 = fire(1, 1)
        put(0, 0, g0)
        put(1, 1, g1)

        def blk(i, carry):
            j = i * 2
            drain_s(0)
            ga = fire(j, 0)
            drain_s(1)
            gb = fire(j + 1, 1)
            put(j, 0, ga)
            put(j + 1, 1, gb)
            return carry

        lax.fori_loop(1, half // 2, blk, 0)
        drain_s(0)
        drain_s(1)

    if with_cnt:
        def cdr(i, carry):
            pltpu.make_async_copy(ones1, onesb, csem).wait()
            return carry

        lax.fori_loop(0, nrows, cdr, 0)
    plsc.subcore_barrier()
    pltpu.sync_copy(acc.at[pl.ds(r0, RPT)],
                    out.at[pl.ds(c * N_PAD + r0, RPT)])
    if with_cnt:
        pltpu.sync_copy(cacc.at[pl.ds(r0, RPT)],
                        cntp.at[pl.ds(c * N_PAD + r0, RPT)])


def _make_agg(with_cnt):
    half = 40
    out_type = [jax.ShapeDtypeStruct((NC * N_PAD, D), jnp.float32)]
    scratch = [
        pltpu.VMEM_SHARED((N_PAD, D), jnp.float32),
        pltpu.VMEM((half, 128), jnp.int32),
        pltpu.VMEM((half, 128), jnp.int32),
        pltpu.VMEM((2, 128, D), jnp.float32),
        pltpu.SemaphoreType.DMA,
        pltpu.SemaphoreType.DMA,
        pltpu.SemaphoreType.DMA,
        pltpu.SemaphoreType.DMA,
    ]
    if with_cnt:
        out_type.append(jax.ShapeDtypeStruct((NC * N_PAD,), jnp.float32))
        scratch = ([pltpu.VMEM_SHARED((N_PAD, D), jnp.float32),
                    pltpu.VMEM_SHARED((N_PAD,), jnp.float32)]
                   + scratch[1:4]
                   + [pltpu.VMEM((128,), jnp.float32)]
                   + scratch[4:]
                   + [pltpu.SemaphoreType.DMA])
    return functools.partial(
        pl.kernel,
        out_type=out_type if with_cnt else out_type[0],
        mesh=plsc.VectorSubcoreMesh(**_MESH),
        scratch_types=scratch,
    )(functools.partial(_agg_body, with_cnt))


def _gather8_body(xs, y1s, z1s, y2s, z2s, y3s, inv0v, invc2v,
                  c1g, c1s, c2p, c3p,
                  oA, oB, oC, oD, oF, oG, oH, oU, idxb, rows, gsem, isem,
                  wsA, wsB):
    c = lax.axis_index("c")
    s = lax.axis_index("s")
    w = c * NS + s
    ncp = GPW // 128
    stg_idx = [c1g, c1s, c2p, c3p]
    stg = []
    for t, idx_ref in enumerate(stg_idx):
        for ch in range(ncp):
            stg.append(pltpu.async_copy(
                idx_ref.at[pl.ds(w * GPW + ch * 128, 128)],
                idxb.at[t * ncp + ch], isem))
    for d in stg:
        d.wait()
    der = []
    for ch in range(ncp):
        der.append(pltpu.async_copy(inv0v.at[idxb.at[2 * ncp + ch]],
                                    idxb.at[4 * ncp + ch], isem))
    for ch in range(ncp):
        der.append(pltpu.async_copy(invc2v.at[idxb.at[3 * ncp + ch]],
                                    idxb.at[5 * ncp + ch], isem))
    for d in der:
        d.wait()
    specs = [(y1s, 0, oA), (z1s, 2, oB), (y2s, 2, oC), (z2s, 3, oD),
             (y3s, 3, oF), (xs, 4, oG), (xs, 5, oH)]
    wdesc = [[], []]
    wsem = (wsA, wsB)
    k = 0
    for src_ref, t, out_ref in specs:
        for ch in range(ncp):
            b = k % 2
            for d in wdesc[b]:
                d.wait()
            wdesc[b] = []
            g = pltpu.async_copy(src_ref.at[idxb.at[t * ncp + ch]],
                                 rows.at[b], gsem)
            g.wait()
            wdesc[b].append(pltpu.async_copy(
                rows.at[b], out_ref.at[pl.ds(w * GPW + ch * 128, 128)],
                wsem[b]))
            if out_ref is oF:
                wdesc[b].append(pltpu.async_copy(
                    rows.at[b], oU.at[idxb.at[1 * ncp + ch]], wsem[b]))
            k += 1
    for b in range(2):
        for d in wdesc[b]:
            d.wait()


def _make_gather8():
    ot = [jax.ShapeDtypeStruct((NG, D), jnp.float32) for _ in range(7)]
    ot.append(jax.ShapeDtypeStruct((N_PAD, D), jnp.float32))
    return functools.partial(
        pl.kernel,
        out_type=ot,
        mesh=plsc.VectorSubcoreMesh(**_MESH),
        scratch_types=[
            pltpu.VMEM((6 * (GPW // 128), 128), jnp.int32),
            pltpu.VMEM((2, 128, D), jnp.float32),
            pltpu.SemaphoreType.DMA,
            pltpu.SemaphoreType.DMA,
            pltpu.SemaphoreType.DMA,
            pltpu.SemaphoreType.DMA,
        ],
    )(_gather8_body)


_BN = 640
_GRID = 16


def _vspec(bn=_BN, d=D):
    return pl.BlockSpec((bn, d), lambda i: (i, 0))


def _wspec():
    return pl.BlockSpec((D, D), lambda i: (0, 0))


def _bspec():
    return pl.BlockSpec((1, D), lambda i: (0, 0))


def _pre0_body(x_ref, z_ref, t_ref):
    xb = x_ref[...]
    t = jnp.mean(xb, axis=1, keepdims=True)
    t_ref[...] = t
    z_ref[...] = xb * jnp.tanh(t)


def _pre0(x):
    return pl.pallas_call(
        _pre0_body,
        grid=(_GRID,),
        in_specs=[_vspec()],
        out_specs=[_vspec(), _vspec(_BN, 1)],
        out_shape=[jax.ShapeDtypeStruct((N, D), jnp.float32),
                   jax.ShapeDtypeStruct((N, 1), jnp.float32)],
    )(x)


def _combine_body(mode, p0_ref, p1_ref, cnt_ref, z_ref, wl_ref, wr_ref,
                  b_ref, *outs):
    den = jnp.maximum(cnt_ref[...], 1.0)
    mean = (p0_ref[...] + p1_ref[...]) / den
    h = (jnp.dot(mean, wl_ref[...], preferred_element_type=jnp.float32,
                 precision=lax.Precision.HIGHEST)
         + jnp.dot(z_ref[...], wr_ref[...], preferred_element_type=jnp.float32,
                   precision=lax.Precision.HIGHEST)
         + b_ref[...])
    if mode == "plain":
        outs[0][...] = h
        return
    y = jnp.maximum(h, 0.0)
    outs[0][...] = y
    if mode == "gate":
        t = jnp.mean(y, axis=1, keepdims=True)
        outs[2][...] = t
        outs[1][...] = y * jnp.tanh(t)


def _combine(mode, sums, cnt, z, wl, wr, b):
    nout = {"gate": 3, "relu": 1, "plain": 1}[mode]
    out_specs = [_vspec(), _vspec(), _vspec(_BN, 1)][:nout]
    out_shape = [jax.ShapeDtypeStruct((N, D), jnp.float32),
                 jax.ShapeDtypeStruct((N, D), jnp.float32),
                 jax.ShapeDtypeStruct((N, 1), jnp.float32)][:nout]
    res = pl.pallas_call(
        functools.partial(_combine_body, mode),
        grid=(_GRID,),
        in_specs=[_vspec(),
                  pl.BlockSpec((_BN, D), lambda i: (i + N_PAD // _BN, 0)),
                  _vspec(_BN, 1), _vspec(),
                  _wspec(), _wspec(), _bspec()],
        out_specs=out_specs,
        out_shape=out_shape,
    )(sums, sums, cnt, z, wl, wr, b.reshape(1, D))
    return res if nout > 1 else res[0]


def _add_body(a_ref, b_ref, o_ref):
    o_ref[...] = a_ref[...] + b_ref[...]


def _add(a, b):
    return pl.pallas_call(
        _add_body,
        grid=(_GRID,),
        in_specs=[_vspec(), _vspec()],
        out_specs=_vspec(),
        out_shape=jax.ShapeDtypeStruct((N, D), jnp.float32),
    )(a, b)


def _loss_body(x_ref, y1_ref, a_ref, b_ref, c_ref, d_ref, f_ref, g_ref,
               h_ref, o_ref):
    i = pl.program_id(0)
    rid = i * _BN + lax.broadcasted_iota(jnp.int32, (_BN, 1), 0)
    m = rid < N
    x, y1 = x_ref[...], y1_ref[...]
    A, B, C, Dv = a_ref[...], b_ref[...], c_ref[...], d_ref[...]
    F, G, H = f_ref[...], g_ref[...], h_ref[...]
    dif = lambda u, v: jnp.where(m, u - v, 0.0)
    sq = lambda u, v: jnp.sum(dif(u, v) ** 2)
    ab = lambda u, v: jnp.sum(jnp.abs(dif(u, v)))
    vals = jnp.stack([sq(A, B), sq(C, Dv), sq(y1, x), sq(C, G), sq(F, H),
                      ab(A, x), ab(C, A), ab(F, C)])

    @pl.when(i == 0)
    def _():
        o_ref[...] = jnp.zeros_like(o_ref)

    o_ref[...] += vals[None, :]


def _losses(x, y1, A, B, C, Dv, F, G, H):
    return pl.pallas_call(
        _loss_body,
        grid=(_GRID,),
        in_specs=[_vspec()] * 9,
        out_specs=pl.BlockSpec((1, 8), lambda i: (0, 0)),
        out_shape=jax.ShapeDtypeStruct((1, 8), jnp.float32),
    )(x, y1, A, B, C, Dv, F, G, H)


def kernel(x, edge_index,
           W_l_0, W_r_0, b_0, W_l_1, W_r_1, b_1, W_l_2, W_r_2, b_2,
           W_l_3, W_r_3, b_3, W_l_4, W_r_4, b_4, W_l_5, W_r_5, b_5):
    E = edge_index.shape[1]
    per_tile = -(-E // (NW * 512)) * 512
    EP = per_tile * NW
    pad = EP - E
    ar = jnp.arange(pad, dtype=jnp.int32)
    srcp = jnp.concatenate([edge_index[0], ar % 32]).reshape(EP // 128, 128)
    dstp = jnp.concatenate([edge_index[1], N + (ar % 8)]).reshape(
        EP // 128, 128)
    zeros2 = jnp.zeros((N_PAD, D), jnp.float32)
    zeros1 = jnp.zeros((N_PAD,), jnp.float32)
    ones1 = jnp.ones((128,), jnp.float32)

    agg = _make_agg(False)

    z0, t0 = _pre0(x)
    sums0, cntp = _make_agg(True)(z0, srcp, dstp, zeros2, zeros1, ones1)
    cnt = (cntp[:N_PAD] + cntp[N_PAD:]).reshape(N_PAD, 1)
    y1, z1, t1 = _combine("gate", sums0, cnt, z0, W_l_0, W_r_0, b_0)
    sums1 = agg(z1, srcp, dstp, zeros2)
    y2, z2, t2 = _combine("gate", sums1, cnt, z1, W_l_1, W_r_1, b_1)
    sums2 = agg(z2, srcp, dstp, zeros2)
    y3 = _combine("relu", sums2, cnt, z2, W_l_2, W_r_2, b_2)

    t0f, t1f, t2f = t0[:, 0], t1[:, 0], t2[:, 0]
    iot = jnp.arange(N, dtype=jnp.int32)
    _, c1 = lax.sort((-t0f, iot), num_keys=1, is_stable=True)
    _, inv0 = lax.sort((c1, iot), num_keys=1, is_stable=True)
    _, _, c2 = lax.sort((-t1f, inv0, iot), num_keys=2, is_stable=True)
    _, invc2 = lax.sort((c2, iot), num_keys=1, is_stable=True)
    _, _, c3 = lax.sort((-t2f, invc2, iot), num_keys=2, is_stable=True)

    npad = NG - N
    pad_g = jnp.arange(npad, dtype=jnp.int32) % 32
    pad_s = N + (jnp.arange(npad, dtype=jnp.int32) % 8)

    def pidx(a, pad):
        return jnp.concatenate([a, pad])

    A, B, C, Dv, F, G, H, Y3U = _make_gather8()(
        x, y1, z1, y2, z2, y3, inv0, invc2,
        pidx(c1, pad_g), pidx(c1, pad_s), pidx(c2, pad_g),
        pidx(c3, pad_g))

    xu = _add(Y3U, x)
    sums3 = agg(xu, srcp, dstp, zeros2)
    y4 = _combine("relu", sums3, cnt, xu, W_l_3, W_r_3, b_3)
    sums4 = agg(y4, srcp, dstp, zeros2)
    out = _combine("plain", sums4, cnt, y4, W_l_5, W_r_5, b_5)

    sums = _losses(x, y1, A, B, C, Dv, F, G, H)
    l = sums[0] / jnp.float32(N * D)
    return (out, l[0], l[1], l[2], l[3], l[4], l[5], l[6], l[7])


# speedup vs baseline: 21.8838x; 1.0141x over previous
import functools

import jax
import jax.numpy as jnp
from jax import lax
from jax.experimental import pallas as pl
from jax.experimental.pallas import tpu as pltpu
from jax.experimental.pallas import tpu_sc as plsc

N = 10000
D = 128
NC, NS = 2, 16
NW = NC * NS
N_PAD = 10240
RPT = N_PAD // NS
NG = 12288
GPW = NG // NW

_MESH = dict(core_axis_name="c", subcore_axis_name="s", num_cores=NC,
             num_subcores=NS)


def _agg_body(with_cnt, *refs):
    if with_cnt:
        (zs, srcs, dsts, zeros2, zeros1, ones1, out, cntp,
         acc, cacc, srcb, dstb, rows2, onesb, gsA, gsB, ssA, ssB,
         csem) = refs
    else:
        (zs, srcs, dsts, zeros2, out,
         acc, srcb, dstb, rows2, gsA, gsB, ssA, ssB) = refs
    c = lax.axis_index("c")
    s = lax.axis_index("s")
    w = c * NS + s
    r0 = s * RPT
    nrows = srcs.shape[0] // NW
    half = nrows // 2
    gs = (gsA, gsB)
    ss = (ssA, ssB)
    initd = pltpu.async_copy(zeros2.at[pl.ds(r0, RPT)],
                             acc.at[pl.ds(r0, RPT)], gsA)
    if with_cnt:
        initc = pltpu.async_copy(zeros1.at[pl.ds(r0, RPT)],
                                 cacc.at[pl.ds(r0, RPT)], gsB)
        pltpu.sync_copy(ones1, onesb)
    pltpu.sync_copy(srcs.at[pl.ds(w * nrows, half)], srcb)
    pltpu.sync_copy(dsts.at[pl.ds(w * nrows, half)], dstb)
    initd.wait()
    if with_cnt:
        initc.wait()
    plsc.subcore_barrier()

    for h in range(2):
        if h > 0:
            pltpu.sync_copy(srcs.at[pl.ds(w * nrows + h * half, half)],
                            srcb)
            pltpu.sync_copy(dsts.at[pl.ds(w * nrows + h * half, half)],
                            dstb)

        def fire(j, b):
            g = pltpu.async_copy(zs.at[srcb.at[j]], rows2.at[b], gs[b])
            if with_cnt:
                pltpu.async_copy(onesb, cacc.at[dstb.at[j]], csem, add=True)
            return g

        def put(j, b, g):
            g.wait()
            pltpu.async_copy(rows2.at[b], acc.at[dstb.at[j]], ss[b],
                             add=True)

        def drain_s(b):
            pltpu.make_async_copy(zeros2.at[pl.ds(0, 128)], rows2.at[b],
                                  ss[b]).wait()

        g0 = fire(0, 0)
        g1 = fire(1, 1)
        put(0, 0, g0)
        put(1, 1, g1)

        def blk(i, carry):
            j = i * 2
            drain_s(0)
            ga = fire(j, 0)
            drain_s(1)
            gb = fire(j + 1, 1)
            put(j, 0, ga)
            put(j + 1, 1, gb)
            return carry

        lax.fori_loop(1, half // 2, blk, 0)
        drain_s(0)
        drain_s(1)

    if with_cnt:
        def cdr(i, carry):
            pltpu.make_async_copy(ones1, onesb, csem).wait()
            return carry

        lax.fori_loop(0, nrows, cdr, 0)
    plsc.subcore_barrier()
    pltpu.sync_copy(acc.at[pl.ds(r0, RPT)],
                    out.at[pl.ds(c * N_PAD + r0, RPT)])
    if with_cnt:
        pltpu.sync_copy(cacc.at[pl.ds(r0, RPT)],
                        cntp.at[pl.ds(c * N_PAD + r0, RPT)])


def _make_agg(with_cnt):
    half = 40
    out_type = [jax.ShapeDtypeStruct((NC * N_PAD, D), jnp.float32)]
    scratch = [
        pltpu.VMEM_SHARED((N_PAD, D), jnp.float32),
        pltpu.VMEM((half, 128), jnp.int32),
        pltpu.VMEM((half, 128), jnp.int32),
        pltpu.VMEM((2, 128, D), jnp.float32),
        pltpu.SemaphoreType.DMA,
        pltpu.SemaphoreType.DMA,
        pltpu.SemaphoreType.DMA,
        pltpu.SemaphoreType.DMA,
    ]
    if with_cnt:
        out_type.append(jax.ShapeDtypeStruct((NC * N_PAD,), jnp.float32))
        scratch = ([pltpu.VMEM_SHARED((N_PAD, D), jnp.float32),
                    pltpu.VMEM_SHARED((N_PAD,), jnp.float32)]
                   + scratch[1:4]
                   + [pltpu.VMEM((128,), jnp.float32)]
                   + scratch[4:]
                   + [pltpu.SemaphoreType.DMA])
    return functools.partial(
        pl.kernel,
        out_type=out_type if with_cnt else out_type[0],
        mesh=plsc.VectorSubcoreMesh(**_MESH),
        scratch_types=scratch,
    )(functools.partial(_agg_body, with_cnt))


def _gather8_body(xs, y1s, z1s, y2s, z2s, y3s, inv0v, invc2v,
                  c1g, c1s, c2p, c3p,
                  oA, oB, oC, oD, oF, oG, oH, oU, idxb, rows, gsem, isem,
                  wsA, wsB):
    c = lax.axis_index("c")
    s = lax.axis_index("s")
    w = c * NS + s
    ncp = GPW // 128
    stg_idx = [c1g, c1s, c2p, c3p]
    stg = []
    for t, idx_ref in enumerate(stg_idx):
        for ch in range(ncp):
            stg.append(pltpu.async_copy(
                idx_ref.at[pl.ds(w * GPW + ch * 128, 128)],
                idxb.at[t * ncp + ch], isem))
    for d in stg:
        d.wait()
    der = []
    for ch in range(ncp):
        der.append(pltpu.async_copy(inv0v.at[idxb.at[2 * ncp + ch]],
                                    idxb.at[4 * ncp + ch], isem))
    for ch in range(ncp):
        der.append(pltpu.async_copy(invc2v.at[idxb.at[3 * ncp + ch]],
                                    idxb.at[5 * ncp + ch], isem))
    for d in der:
        d.wait()
    specs = [(y1s, 0, oA), (z1s, 2, oB), (y2s, 2, oC), (z2s, 3, oD),
             (y3s, 3, oF), (xs, 4, oG), (xs, 5, oH)]
    wdesc = [[], []]
    wsem = (wsA, wsB)
    k = 0
    for src_ref, t, out_ref in specs:
        for ch in range(ncp):
            b = k % 2
            for d in wdesc[b]:
                d.wait()
            wdesc[b] = []
            g = pltpu.async_copy(src_ref.at[idxb.at[t * ncp + ch]],
                                 rows.at[b], gsem)
            g.wait()
            wdesc[b].append(pltpu.async_copy(
                rows.at[b], out_ref.at[pl.ds(w * GPW + ch * 128, 128)],
                wsem[b]))
            if out_ref is oF:
                wdesc[b].append(pltpu.async_copy(
                    rows.at[b], oU.at[idxb.at[1 * ncp + ch]], wsem[b]))
            k += 1
    for b in range(2):
        for d in wdesc[b]:
            d.wait()


def _make_gather8():
    ot = [jax.ShapeDtypeStruct((NG, D), jnp.float32) for _ in range(7)]
    ot.append(jax.ShapeDtypeStruct((N_PAD, D), jnp.float32))
    return functools.partial(
        pl.kernel,
        out_type=ot,
        mesh=plsc.VectorSubcoreMesh(**_MESH),
        scratch_types=[
            pltpu.VMEM((6 * (GPW // 128), 128), jnp.int32),
            pltpu.VMEM((2, 128, D), jnp.float32),
            pltpu.SemaphoreType.DMA,
            pltpu.SemaphoreType.DMA,
            pltpu.SemaphoreType.DMA,
            pltpu.SemaphoreType.DMA,
        ],
    )(_gather8_body)


_BN = 640
_GRID = 16


def _vspec(bn=_BN, d=D):
    return pl.BlockSpec((bn, d), lambda i: (i, 0))


def _wspec():
    return pl.BlockSpec((D, D), lambda i: (0, 0))


def _bspec():
    return pl.BlockSpec((1, D), lambda i: (0, 0))


def _pre0_body(x_ref, z_ref, t_ref):
    xb = x_ref[...]
    t = jnp.mean(xb, axis=1, keepdims=True)
    t_ref[...] = t
    z_ref[...] = xb * jnp.tanh(t)


def _pre0(x):
    return pl.pallas_call(
        _pre0_body,
        grid=(_GRID,),
        in_specs=[_vspec()],
        out_specs=[_vspec(), _vspec(_BN, 1)],
        out_shape=[jax.ShapeDtypeStruct((N, D), jnp.float32),
                   jax.ShapeDtypeStruct((N, 1), jnp.float32)],
    )(x)


def _combine_body(mode, p0_ref, p1_ref, cnt_ref, z_ref, wl_ref, wr_ref,
                  b_ref, *outs):
    den = jnp.maximum(cnt_ref[...], 1.0)
    mean = (p0_ref[...] + p1_ref[...]) / den
    h = (jnp.dot(mean, wl_ref[...], preferred_element_type=jnp.float32,
                 precision=lax.Precision.HIGHEST)
         + jnp.dot(z_ref[...], wr_ref[...], preferred_element_type=jnp.float32,
                   precision=lax.Precision.HIGHEST)
         + b_ref[...])
    if mode == "plain":
        outs[0][...] = h
        return
    y = jnp.maximum(h, 0.0)
    outs[0][...] = y
    if mode == "gate":
        t = jnp.mean(y, axis=1, keepdims=True)
        outs[2][...] = t
        outs[1][...] = y * jnp.tanh(t)


def _combine(mode, sums, cnt, z, wl, wr, b):
    nout = {"gate": 3, "relu": 1, "plain": 1}[mode]
    out_specs = [_vspec(), _vspec(), _vspec(_BN, 1)][:nout]
    out_shape = [jax.ShapeDtypeStruct((N, D), jnp.float32),
                 jax.ShapeDtypeStruct((N, D), jnp.float32),
                 jax.ShapeDtypeStruct((N, 1), jnp.float32)][:nout]
    res = pl.pallas_call(
        functools.partial(_combine_body, mode),
        grid=(_GRID,),
        in_specs=[_vspec(),
                  pl.BlockSpec((_BN, D), lambda i: (i + N_PAD // _BN, 0)),
                  _vspec(_BN, 1), _vspec(),
                  _wspec(), _wspec(), _bspec()],
        out_specs=out_specs,
        out_shape=out_shape,
    )(sums, sums, cnt, z, wl, wr, b.reshape(1, D))
    return res if nout > 1 else res[0]


def _add_body(a_ref, b_ref, o_ref):
    o_ref[...] = a_ref[...] + b_ref[...]


def _add(a, b):
    return pl.pallas_call(
        _add_body,
        grid=(_GRID,),
        in_specs=[_vspec(), _vspec()],
        out_specs=_vspec(),
        out_shape=jax.ShapeDtypeStruct((N, D), jnp.float32),
    )(a, b)


def _loss_body(x_ref, y1_ref, a_ref, b_ref, c_ref, d_ref, f_ref, g_ref,
               h_ref, o_ref):
    i = pl.program_id(0)
    rid = i * _BN + lax.broadcasted_iota(jnp.int32, (_BN, 1), 0)
    m = rid < N
    x, y1 = x_ref[...], y1_ref[...]
    A, B, C, Dv = a_ref[...], b_ref[...], c_ref[...], d_ref[...]
    F, G, H = f_ref[...], g_ref[...], h_ref[...]
    dif = lambda u, v: jnp.where(m, u - v, 0.0)
    sq = lambda u, v: jnp.sum(dif(u, v) ** 2)
    ab = lambda u, v: jnp.sum(jnp.abs(dif(u, v)))
    vals = jnp.stack([sq(A, B), sq(C, Dv), sq(y1, x), sq(C, G), sq(F, H),
                      ab(A, x), ab(C, A), ab(F, C)])

    @pl.when(i == 0)
    def _():
        o_ref[...] = jnp.zeros_like(o_ref)

    o_ref[...] += vals[None, :]


def _losses(x, y1, A, B, C, Dv, F, G, H):
    return pl.pallas_call(
        _loss_body,
        grid=(_GRID,),
        in_specs=[_vspec()] * 9,
        out_specs=pl.BlockSpec((1, 8), lambda i: (0, 0)),
        out_shape=jax.ShapeDtypeStruct((1, 8), jnp.float32),
    )(x, y1, A, B, C, Dv, F, G, H)


def kernel(x, edge_index,
           W_l_0, W_r_0, b_0, W_l_1, W_r_1, b_1, W_l_2, W_r_2, b_2,
           W_l_3, W_r_3, b_3, W_l_4, W_r_4, b_4, W_l_5, W_r_5, b_5):
    E = edge_index.shape[1]
    per_tile = -(-E // (NW * 512)) * 512
    EP = per_tile * NW
    pad = EP - E
    ar = jnp.arange(pad, dtype=jnp.int32)
    srcp = jnp.concatenate([edge_index[0], ar % 32]).reshape(EP // 128, 128)
    dstp = jnp.concatenate([edge_index[1], N + (ar % 8)]).reshape(
        EP // 128, 128)
    zeros2 = jnp.zeros((N_PAD, D), jnp.float32)
    zeros1 = jnp.zeros((N_PAD,), jnp.float32)
    ones1 = jnp.ones((128,), jnp.float32)

    agg = _make_agg(False)

    z0, t0 = _pre0(x)
    sums0, cntp = _make_agg(True)(z0, srcp, dstp, zeros2, zeros1, ones1)
    cnt = (cntp[:N_PAD] + cntp[N_PAD:]).reshape(N_PAD, 1)
    y1, z1, t1 = _combine("gate", sums0, cnt, z0, W_l_0, W_r_0, b_0)
    sums1 = agg(z1, srcp, dstp, zeros2)
    y2, z2, t2 = _combine("gate", sums1, cnt, z1, W_l_1, W_r_1, b_1)
    sums2 = agg(z2, srcp, dstp, zeros2)
    y3 = _combine("relu", sums2, cnt, z2, W_l_2, W_r_2, b_2)

    t0f, t1f, t2f = t0[:, 0], t1[:, 0], t2[:, 0]
    iot = jnp.arange(N, dtype=jnp.int32)
    _, c1 = lax.sort((-t0f, iot), num_keys=1, is_stable=True)
    _, inv0 = lax.sort((c1, iot), num_keys=1, is_stable=True)
    _, _, c2 = lax.sort((-t1f, inv0, iot), num_keys=2, is_stable=True)
    _, invc2 = lax.sort((c2, iot), num_keys=1, is_stable=True)
    _, _, c3 = lax.sort((-t2f, invc2, iot), num_keys=2, is_stable=True)

    npad = NG - N
    pad_g = jnp.arange(npad, dtype=jnp.int32) % 32
    pad_s = N + (jnp.arange(npad, dtype=jnp.int32) % 8)

    def pidx(a, pad):
        return jnp.concatenate([a, pad])

    A, B, C, Dv, F, G, H, Y3U = _make_gather8()(
        x, y1, z1, y2, z2, y3, inv0, invc2,
        pidx(c1, pad_g), pidx(c1, pad_s), pidx(c2, pad_g),
        pidx(c3, pad_g))

    xu = _add(Y3U, x)
    sums3 = agg(xu, srcp, dstp, zeros2)
    y4 = _combine("relu", sums3, cnt, xu, W_l_3, W_r_3, b_3)
    sums4 = agg(y4, srcp, dstp, zeros2)
    out = _combine("plain", sums4, cnt, y4, W_l_5, W_r_5, b_5)

    sums = _losses(x, y1, A, B, C, Dv, F, G, H)
    l = sums[0] / jnp.float32(N * D)
    return (out, l[0], l[1], l[2], l[3], l[4], l[5], l[6], l[7])
